# baseline (speedup 1.0000x reference)
import jax
import jax.numpy as jnp
from jax import lax
from jax.experimental import pallas as pl
from jax.experimental.pallas import tpu as pltpu
from jax.experimental.pallas import tpu_sc as plsc

_NC = 2
_NS = 16
_NW = _NC * _NS
_L = 16
_BINS = 10

_N = 2_000_000
_BLK = 128
_VPB = _BLK // _L
_NBLK = _N // _BLK
_BPT = _NBLK // _NW
_RBLK = _NBLK - _BPT * _NW
_CHUNK_B = 122
_NCHUNK = _BPT // _CHUNK_B
_CHUNK_S = _CHUNK_B * _BLK


def _sc_body(feat_hbm, lab_hbm, cen_hbm, out_hbm,
             lab0_v, lab1_v, feat0_v, feat1_v, cen_v, cnt_t, sq_t,
             fcnt_t, fsq_t, rlab_v, rfeat_v, sem0, sem1):
    wid = lax.axis_index("s") * _NC + lax.axis_index("c")

    iota = lax.iota(jnp.int32, _L)
    ones_i = jnp.ones((_L,), jnp.int32)
    ones_f = jnp.ones((_L,), jnp.float32)
    zeros_f = jnp.zeros((_L,), jnp.float32)

    for i in range(_BINS * _VPB):
        cnt_t[pl.ds(i * _L, _L)] = zeros_f
        sq_t[pl.ds(i * _L, _L)] = zeros_f

    pltpu.sync_copy(cen_hbm, cen_v)

    labs = (lab0_v, lab1_v)
    feats = (feat0_v, feat1_v)
    sems = (sem0, sem1)

    def vec_body(sbase, p0, rep, lab_ref, feat_ref):
        lab = lab_ref[pl.ds(sbase, _L)]
        li = lab.astype(jnp.int32)
        f0 = feat_ref[pl.ds(p0, _L)]
        f1 = feat_ref[pl.ds(p0 + _BLK, _L)]
        ci = 2 * li
        c0 = plsc.load_gather(cen_v, [ci])
        c1 = plsc.load_gather(cen_v, [ci + ones_i])
        d0 = f0 - c0
        d1 = f1 - c1
        sq = d0 * d0 + d1 * d1
        idx = li * _L + iota + (rep * _BINS * _L)
        plsc.addupdate_scatter(cnt_t, [idx], ones_f)
        plsc.addupdate_scatter(sq_t, [idx], sq)

    tile_base = wid * (_BPT * _BLK)

    def start_chunk(b, ci):
        s0 = pl.multiple_of(tile_base + ci * _CHUNK_S, _BLK)
        return (
            pltpu.async_copy(lab_hbm.at[pl.ds(s0, _CHUNK_S)], labs[b], sems[b]),
            pltpu.async_copy(
                feat_hbm.at[pl.ds(2 * s0, 2 * _CHUNK_S)], feats[b], sems[b]),
        )

    def block_loop(lab_ref, feat_ref, nblk):
        @plsc.parallel_loop(0, nblk, unroll=2, carry=jnp.int32(0))
        def blk(q, c):
            qb = pl.multiple_of(q * _BLK, _BLK)
            q2 = pl.multiple_of(q * (2 * _BLK), 2 * _BLK)
            for r in range(_VPB):
                vec_body(qb + r * _L, q2 + r * _L, r, lab_ref, feat_ref)
            return c

    pending = start_chunk(0, 0)
    for ci in range(_NCHUNK):
        b = ci % 2
        for d in pending:
            d.wait()
        if ci + 1 < _NCHUNK:
            pending = start_chunk(1 - b, ci + 1)
        block_loop(labs[b], feats[b], _CHUNK_B)

    @pl.when(wid < _RBLK)
    def _():
        r0 = pl.multiple_of((_BPT * _NW + wid) * _BLK, _BLK)
        pltpu.sync_copy(lab_hbm.at[pl.ds(r0, _BLK)], rlab_v)
        pltpu.sync_copy(feat_hbm.at[pl.ds(2 * r0, 2 * _BLK)], rfeat_v)
        for r in range(_VPB):
            vec_body(r * _L, r * _L, r, rlab_v, rfeat_v)

    for i in range(_BINS):
        acc_c = cnt_t[pl.ds(i * _L, _L)]
        acc_s = sq_t[pl.ds(i * _L, _L)]
        for rep in range(1, _VPB):
            off = rep * _BINS * _L + i * _L
            acc_c = acc_c + cnt_t[pl.ds(off, _L)]
            acc_s = acc_s + sq_t[pl.ds(off, _L)]
        fcnt_t[pl.ds(i * _L, _L)] = acc_c
        fsq_t[pl.ds(i * _L, _L)] = acc_s

    pltpu.sync_copy(fcnt_t, out_hbm.at[wid, 0])
    pltpu.sync_copy(fsq_t, out_hbm.at[wid, 1])


@jax.jit
def _centerloss(feature, label, lambdas, center):
    feat_flat = feature.reshape(_NBLK, _BLK, 2).transpose(0, 2, 1).reshape(-1)
    mesh = plsc.VectorSubcoreMesh(core_axis_name="c", subcore_axis_name="s")
    part = pl.kernel(
        _sc_body,
        out_type=jax.ShapeDtypeStruct((_NW, 2, _BINS * _L), jnp.float32),
        mesh=mesh,
        scratch_types=[
            pltpu.VMEM((_CHUNK_S,), jnp.float32),
            pltpu.VMEM((_CHUNK_S,), jnp.float32),
            pltpu.VMEM((2 * _CHUNK_S,), jnp.float32),
            pltpu.VMEM((2 * _CHUNK_S,), jnp.float32),
            pltpu.VMEM((2 * _BINS,), jnp.float32),
            pltpu.VMEM((_VPB * _BINS * _L,), jnp.float32),
            pltpu.VMEM((_VPB * _BINS * _L,), jnp.float32),
            pltpu.VMEM((_BINS * _L,), jnp.float32),
            pltpu.VMEM((_BINS * _L,), jnp.float32),
            pltpu.VMEM((_BLK,), jnp.float32),
            pltpu.VMEM((2 * _BLK,), jnp.float32),
            pltpu.SemaphoreType.DMA,
            pltpu.SemaphoreType.DMA,
        ],
        compiler_params=pltpu.CompilerParams(needs_layout_passes=False),
    )(feat_flat, label, center.reshape(-1))

    sub = part.reshape(_NW, 2, _BINS, _L)
    cnt = sub[:, 0].sum(axis=(0, 2))
    ssq = sub[:, 1].sum(axis=(0, 2))
    per_class = jnp.where(cnt > 0, ssq / cnt, jnp.float32(0))
    return lambdas * 0.5 * jnp.sum(per_class) / jnp.float32(_N)


def kernel(feature, label, lambdas, center):
    return _centerloss(feature, label, lambdas, center)


# speedup vs baseline: 1.0180x; 1.0180x over previous
import jax
import jax.numpy as jnp
from jax import lax
from jax.experimental import pallas as pl
from jax.experimental.pallas import tpu as pltpu
from jax.experimental.pallas import tpu_sc as plsc

_NC = 2
_NS = 16
_NW = _NC * _NS
_L = 16
_BINS = 10

_N = 2_000_000
_BLK = 128
_VPB = _BLK // _L
_NBLK = _N // _BLK
_BPT = _NBLK // _NW
_RBLK = _NBLK - _BPT * _NW
_CHUNK_B = 122
_NCHUNK = _BPT // _CHUNK_B
_CHUNK_S = _CHUNK_B * _BLK


def _sc_body(feat_hbm, lab_hbm, out_hbm,
             lab0_v, lab1_v, feat0_v, feat1_v,
             cnt_t, sf0_t, sf1_t, ssq_t, fold_t,
             rlab_v, rfeat_v, sem0, sem1):
    wid = lax.axis_index("s") * _NC + lax.axis_index("c")

    iota = lax.iota(jnp.int32, _L)
    ones_i = jnp.ones((_L,), jnp.int32)
    ones_f = jnp.ones((_L,), jnp.float32)
    zeros_f = jnp.zeros((_L,), jnp.float32)

    for i in range(_BINS * _VPB):
        cnt_t[pl.ds(i * _L, _L)] = zeros_f
        sf0_t[pl.ds(i * _L, _L)] = zeros_f
        sf1_t[pl.ds(i * _L, _L)] = zeros_f
        ssq_t[pl.ds(i * _L, _L)] = zeros_f

    labs = (lab0_v, lab1_v)
    feats = (feat0_v, feat1_v)
    sems = (sem0, sem1)

    def vec_body(sbase, p0, rep, lab_ref, feat_ref):
        lab = lab_ref[pl.ds(sbase, _L)]
        li = lab.astype(jnp.int32)
        f0 = feat_ref[pl.ds(p0, _L)]
        f1 = feat_ref[pl.ds(p0 + _BLK, _L)]
        sq = f0 * f0 + f1 * f1
        idx = li * _L + iota + (rep * _BINS * _L)
        plsc.addupdate_scatter(cnt_t, [idx], ones_f)
        plsc.addupdate_scatter(sf0_t, [idx], f0)
        plsc.addupdate_scatter(sf1_t, [idx], f1)
        plsc.addupdate_scatter(ssq_t, [idx], sq)

    tile_base = wid * (_BPT * _BLK)

    def start_chunk(b, ci):
        s0 = pl.multiple_of(tile_base + ci * _CHUNK_S, _BLK)
        return (
            pltpu.async_copy(lab_hbm.at[pl.ds(s0, _CHUNK_S)], labs[b], sems[b]),
            pltpu.async_copy(
                feat_hbm.at[pl.ds(2 * s0, 2 * _CHUNK_S)], feats[b], sems[b]),
        )

    def block_loop(lab_ref, feat_ref, nblk):
        @plsc.parallel_loop(0, nblk, carry=jnp.int32(0))
        def blk(q, c):
            qb = pl.multiple_of(q * _BLK, _BLK)
            q2 = pl.multiple_of(q * (2 * _BLK), 2 * _BLK)
            for r in range(_VPB):
                vec_body(qb + r * _L, q2 + r * _L, r, lab_ref, feat_ref)
            return c

    pending = start_chunk(0, 0)
    for ci in range(_NCHUNK):
        b = ci % 2
        for d in pending:
            d.wait()
        if ci + 1 < _NCHUNK:
            pending = start_chunk(1 - b, ci + 1)
        block_loop(labs[b], feats[b], _CHUNK_B)

    @pl.when(wid < _RBLK)
    def _():
        r0 = pl.multiple_of((_BPT * _NW + wid) * _BLK, _BLK)
        pltpu.sync_copy(lab_hbm.at[pl.ds(r0, _BLK)], rlab_v)
        pltpu.sync_copy(feat_hbm.at[pl.ds(2 * r0, 2 * _BLK)], rfeat_v)
        for r in range(_VPB):
            vec_body(r * _L, r * _L, r, rlab_v, rfeat_v)

    for t, tab in enumerate((cnt_t, sf0_t, sf1_t, ssq_t)):
        for i in range(_BINS):
            acc = tab[pl.ds(i * _L, _L)]
            for rep in range(1, _VPB):
                acc = acc + tab[pl.ds(rep * _BINS * _L + i * _L, _L)]
            fold_t[pl.ds(t * _BINS * _L + i * _L, _L)] = acc
    pltpu.sync_copy(fold_t, out_hbm.at[wid])


@jax.jit
def _centerloss(feature, label, lambdas, center):
    feat_flat = feature.reshape(_NBLK, _BLK, 2).transpose(0, 2, 1).reshape(-1)
    mesh = plsc.VectorSubcoreMesh(core_axis_name="c", subcore_axis_name="s")
    part = pl.kernel(
        _sc_body,
        out_type=jax.ShapeDtypeStruct((_NW, 4 * _BINS * _L), jnp.float32),
        mesh=mesh,
        scratch_types=[
            pltpu.VMEM((_CHUNK_S,), jnp.float32),
            pltpu.VMEM((_CHUNK_S,), jnp.float32),
            pltpu.VMEM((2 * _CHUNK_S,), jnp.float32),
            pltpu.VMEM((2 * _CHUNK_S,), jnp.float32),
            pltpu.VMEM((_VPB * _BINS * _L,), jnp.float32),
            pltpu.VMEM((_VPB * _BINS * _L,), jnp.float32),
            pltpu.VMEM((_VPB * _BINS * _L,), jnp.float32),
            pltpu.VMEM((_VPB * _BINS * _L,), jnp.float32),
            pltpu.VMEM((4 * _BINS * _L,), jnp.float32),
            pltpu.VMEM((_BLK,), jnp.float32),
            pltpu.VMEM((2 * _BLK,), jnp.float32),
            pltpu.SemaphoreType.DMA,
            pltpu.SemaphoreType.DMA,
        ],
        compiler_params=pltpu.CompilerParams(needs_layout_passes=False),
    )(feat_flat, label)

    sub = part.reshape(_NW, 4, _BINS, _L).sum(axis=(0, 3))
    cnt, sf0, sf1, ssq = sub[0], sub[1], sub[2], sub[3]
    c0 = center[:, 0]
    c1 = center[:, 1]
    s_c = ssq - 2.0 * (c0 * sf0 + c1 * sf1) + cnt * (c0 * c0 + c1 * c1)
    per_class = jnp.where(cnt > 0, s_c / cnt, jnp.float32(0))
    return lambdas * 0.5 * jnp.sum(per_class) / jnp.float32(_N)


def kernel(feature, label, lambdas, center):
    return _centerloss(feature, label, lambdas, center)
